# baseline (speedup 1.0000x reference)
import functools

import jax
import jax.numpy as jnp
from jax import lax
from jax.experimental import pallas as pl
from jax.experimental.pallas import tpu as pltpu
from jax.experimental.pallas import tpu_sc as plsc

N = 10000
E = 320000
D = 128
DC = 128
NC = 2
NS = 16
NW = NC * NS
EPT = E // NW
CH = 128
EPTP = 10240
NCHUNK = EPTP // CH
NP = 10240
PAD_DST = 10100
RPT = NP // NS

_mesh = plsc.VectorSubcoreMesh(core_axis_name="c", subcore_axis_name="s")


@functools.partial(
    pl.kernel,
    out_type=jax.ShapeDtypeStruct((NC, NP, D), jnp.float32),
    mesh=_mesh,
    scratch_types=[
        pltpu.VMEM_SHARED((NP, D), jnp.float32),
        pltpu.VMEM((CH,), jnp.int32),
        pltpu.VMEM((CH,), jnp.int32),
        pltpu.VMEM((CH,), jnp.int32),
        pltpu.VMEM((CH,), jnp.int32),
        pltpu.VMEM((CH, D), jnp.float32),
        pltpu.VMEM((CH, D), jnp.float32),
        pltpu.SemaphoreType.DMA,
        pltpu.SemaphoreType.DMA,
        pltpu.SemaphoreType.DMA,
        pltpu.SemaphoreType.DMA,
        pltpu.SemaphoreType.DMA,
        pltpu.SemaphoreType.DMA,
    ],
)
def _sc_agg(g_hbm, src_hbm, dst_hbm, zeros_hbm, out_hbm,
            acc, sidx0, sidx1, didx0, didx1, rows0, rows1,
            ssi0, ssi1, sdi0, sdi1, sg0, sg1):
    c = lax.axis_index("c")
    s = lax.axis_index("s")
    t = c * NS + s

    def load(hbm, k, ref, sem):
        pltpu.async_copy(hbm.at[t, k], ref, sem)

    def wait(hbm, k, ref, sem):
        pltpu.make_async_copy(hbm.at[t, k], ref, sem).wait()

    load(src_hbm, 0, sidx0, ssi0)
    load(dst_hbm, 0, didx0, sdi0)
    load(src_hbm, 1, sidx1, ssi1)
    load(dst_hbm, 1, didx1, sdi1)
    pltpu.sync_copy(zeros_hbm.at[pl.ds(s * RPT, RPT)],
                    acc.at[pl.ds(s * RPT, RPT)])
    plsc.subcore_barrier()
    wait(src_hbm, 0, sidx0, ssi0)
    pltpu.async_copy(g_hbm.at[sidx0], rows0, sg0)

    def body(j, carry):
        a = 2 * j
        wait(src_hbm, a + 1, sidx1, ssi1)
        wait(dst_hbm, a, didx0, sdi0)
        pltpu.make_async_copy(g_hbm.at[sidx0], rows0, sg0).wait()
        pltpu.sync_copy(rows0, acc.at[didx0], add=True)
        load(src_hbm, a + 2, sidx0, ssi0)
        load(dst_hbm, a + 2, didx0, sdi0)
        pltpu.async_copy(g_hbm.at[sidx1], rows1, sg1)
        wait(src_hbm, a + 2, sidx0, ssi0)
        wait(dst_hbm, a + 1, didx1, sdi1)
        pltpu.make_async_copy(g_hbm.at[sidx1], rows1, sg1).wait()
        pltpu.sync_copy(rows1, acc.at[didx1], add=True)
        load(src_hbm, a + 3, sidx1, ssi1)
        load(dst_hbm, a + 3, didx1, sdi1)
        pltpu.async_copy(g_hbm.at[sidx0], rows0, sg0)
        return carry

    lax.fori_loop(0, NCHUNK // 2 - 1, body, 0)
    a = NCHUNK - 2
    wait(src_hbm, a + 1, sidx1, ssi1)
    wait(dst_hbm, a, didx0, sdi0)
    pltpu.make_async_copy(g_hbm.at[sidx0], rows0, sg0).wait()
    pltpu.sync_copy(rows0, acc.at[didx0], add=True)
    pltpu.async_copy(g_hbm.at[sidx1], rows1, sg1)
    wait(dst_hbm, a + 1, didx1, sdi1)
    pltpu.make_async_copy(g_hbm.at[sidx1], rows1, sg1).wait()
    pltpu.sync_copy(rows1, acc.at[didx1], add=True)

    plsc.subcore_barrier()
    pltpu.sync_copy(acc.at[pl.ds(s * RPT, RPT)],
                    out_hbm.at[c].at[pl.ds(s * RPT, RPT)])


@functools.partial(
    pl.kernel,
    out_type=jax.ShapeDtypeStruct((NC, NP, DC), jnp.float32),
    mesh=_mesh,
    scratch_types=[
        pltpu.VMEM_SHARED((NP, DC), jnp.float32),
        pltpu.VMEM((NCHUNK, CH), jnp.int32),
        pltpu.VMEM((CH, DC), jnp.float32),
        pltpu.SemaphoreType.DMA,
        pltpu.SemaphoreType.DMA,
    ],
)
def _sc_deg(dst_hbm, zeros_hbm, ones_hbm, out_hbm, acc, didx, ones, s0, s1):
    c = lax.axis_index("c")
    s = lax.axis_index("s")
    t = c * NS + s
    pltpu.sync_copy(zeros_hbm.at[pl.ds(s * RPT, RPT)],
                    acc.at[pl.ds(s * RPT, RPT)])
    pltpu.sync_copy(dst_hbm.at[t], didx)
    pltpu.sync_copy(ones_hbm, ones)
    plsc.subcore_barrier()

    def body(j, carry):
        a = 2 * j
        pltpu.async_copy(ones, acc.at[didx.at[a]], s0, add=True)
        pltpu.async_copy(ones, acc.at[didx.at[a + 1]], s1, add=True)
        pltpu.make_async_copy(ones, acc.at[didx.at[a]], s0).wait()
        pltpu.make_async_copy(ones, acc.at[didx.at[a + 1]], s1).wait()
        return carry

    lax.fori_loop(0, NCHUNK // 2, body, 0)
    plsc.subcore_barrier()
    pltpu.sync_copy(acc.at[pl.ds(s * RPT, RPT)],
                    out_hbm.at[c].at[pl.ds(s * RPT, RPT)])


_RB = 1000


def _tc0_body(degp_ref, x_ref, w_ref, dinv_ref, g_ref):
    deg = degp_ref[0, :, 0:1] + degp_ref[1, :, 0:1] + 1.0
    dinv = lax.rsqrt(deg)
    dinv_ref[...] = dinv
    g_ref[...] = dinv * jnp.dot(x_ref[...], w_ref[...],
                                preferred_element_type=jnp.float32)


def _tc_mid_body(aggp_ref, g_ref, dinv_ref, b_ref, w_ref, gout_ref):
    dinv = dinv_ref[...]
    h = dinv * (aggp_ref[0] + aggp_ref[1] + g_ref[...]) + b_ref[...]
    h = jnp.maximum(h, 0.0)
    gout_ref[...] = dinv * jnp.dot(h, w_ref[...],
                                   preferred_element_type=jnp.float32)


def _tc_fin_body(aggp_ref, g_ref, dinv_ref, b_ref, out_ref):
    out_ref[...] = (dinv_ref[...] * (aggp_ref[0] + aggp_ref[1] + g_ref[...])
                    + b_ref[...])


def _tc0(degp, x, w):
    return pl.pallas_call(
        _tc0_body,
        grid=(N // _RB,),
        in_specs=[
            pl.BlockSpec((NC, _RB, DC), lambda i: (0, i, 0)),
            pl.BlockSpec((_RB, D), lambda i: (i, 0)),
            pl.BlockSpec((D, D), lambda i: (0, 0)),
        ],
        out_specs=[
            pl.BlockSpec((_RB, 1), lambda i: (i, 0)),
            pl.BlockSpec((_RB, D), lambda i: (i, 0)),
        ],
        out_shape=[
            jax.ShapeDtypeStruct((N, 1), jnp.float32),
            jax.ShapeDtypeStruct((N, D), jnp.float32),
        ],
    )(degp, x, w)


def _tc_mid(aggp, g, dinv, b, w):
    return pl.pallas_call(
        _tc_mid_body,
        grid=(N // _RB,),
        in_specs=[
            pl.BlockSpec((NC, _RB, D), lambda i: (0, i, 0)),
            pl.BlockSpec((_RB, D), lambda i: (i, 0)),
            pl.BlockSpec((_RB, 1), lambda i: (i, 0)),
            pl.BlockSpec((1, D), lambda i: (0, 0)),
            pl.BlockSpec((D, D), lambda i: (0, 0)),
        ],
        out_specs=pl.BlockSpec((_RB, D), lambda i: (i, 0)),
        out_shape=jax.ShapeDtypeStruct((N, D), jnp.float32),
    )(aggp, g, dinv, b, w)


def _tc_fin(aggp, g, dinv, b):
    return pl.pallas_call(
        _tc_fin_body,
        grid=(N // _RB,),
        in_specs=[
            pl.BlockSpec((NC, _RB, D), lambda i: (0, i, 0)),
            pl.BlockSpec((_RB, D), lambda i: (i, 0)),
            pl.BlockSpec((_RB, 1), lambda i: (i, 0)),
            pl.BlockSpec((1, D), lambda i: (0, 0)),
        ],
        out_specs=pl.BlockSpec((_RB, D), lambda i: (i, 0)),
        out_shape=jax.ShapeDtypeStruct((N, D), jnp.float32),
    )(aggp, g, dinv, b)


def kernel(x, edge_index, edge_features, W1, b1, W2, b2, W3, b3):
    del edge_features
    src = jnp.pad(edge_index[0].reshape(NW, EPT), ((0, 0), (0, EPTP - EPT)),
                  constant_values=0).reshape(NW, NCHUNK, CH)
    dst = jnp.pad(edge_index[1].reshape(NW, EPT), ((0, 0), (0, EPTP - EPT)),
                  constant_values=PAD_DST).reshape(NW, NCHUNK, CH)
    zeros_d = jnp.zeros((NP, D), jnp.float32)
    zeros_dc = jnp.zeros((NP, DC), jnp.float32)
    ones_dc = jnp.ones((CH, DC), jnp.float32)

    degp = _sc_deg(dst, zeros_dc, ones_dc)
    dinv, g1 = _tc0(degp, x, W1)
    agg1 = _sc_agg(g1, src, dst, zeros_d)
    g2 = _tc_mid(agg1, g1, dinv, b1.reshape(1, D), W2)
    agg2 = _sc_agg(g2, src, dst, zeros_d)
    g3 = _tc_mid(agg2, g2, dinv, b2.reshape(1, D), W3)
    agg3 = _sc_agg(g3, src, dst, zeros_d)
    return _tc_fin(agg3, g3, dinv, b3.reshape(1, D))


# speedup vs baseline: 1.9116x; 1.9116x over previous
import functools

import jax
import jax.numpy as jnp
from jax import lax
from jax.experimental import pallas as pl
from jax.experimental.pallas import tpu as pltpu
from jax.experimental.pallas import tpu_sc as plsc

N = 10000
E = 320000
D = 128
DC = 128
NC = 2
NS = 16
NW = NC * NS
EPT = E // NW
CH = 80
EPTP = 10080
NCHUNK = EPTP // CH
NP = 10240
PAD_DST = 10100
RPT = NP // NS

_mesh = plsc.VectorSubcoreMesh(core_axis_name="c", subcore_axis_name="s")


@functools.partial(
    pl.kernel,
    out_type=jax.ShapeDtypeStruct((NC, NP, D), jnp.float32),
    mesh=_mesh,
    scratch_types=[
        pltpu.VMEM_SHARED((NP, D), jnp.float32),
        pltpu.VMEM((CH,), jnp.int32),
        pltpu.VMEM((CH,), jnp.int32),
        pltpu.VMEM((CH,), jnp.int32),
        pltpu.VMEM((CH,), jnp.int32),
        pltpu.VMEM((CH, D), jnp.float32),
        pltpu.VMEM((CH, D), jnp.float32),
        pltpu.SemaphoreType.DMA,
        pltpu.SemaphoreType.DMA,
        pltpu.SemaphoreType.DMA,
        pltpu.SemaphoreType.DMA,
        pltpu.SemaphoreType.DMA,
        pltpu.SemaphoreType.DMA,
    ],
)
def _sc_agg(g_hbm, src_hbm, dst_hbm, zeros_hbm, out_hbm,
            acc, sidx0, sidx1, didx0, didx1, rows0, rows1,
            ssi0, ssi1, sdi0, sdi1, sg0, sg1):
    c = lax.axis_index("c")
    s = lax.axis_index("s")
    t = c * NS + s

    def load(hbm, k, ref, sem):
        pltpu.async_copy(hbm.at[t, k], ref, sem)

    def wait(hbm, k, ref, sem):
        pltpu.make_async_copy(hbm.at[t, k], ref, sem).wait()

    load(src_hbm, 0, sidx0, ssi0)
    load(dst_hbm, 0, didx0, sdi0)
    load(src_hbm, 1, sidx1, ssi1)
    load(dst_hbm, 1, didx1, sdi1)
    pltpu.sync_copy(zeros_hbm.at[pl.ds(s * RPT, RPT)],
                    acc.at[pl.ds(s * RPT, RPT)])
    plsc.subcore_barrier()
    wait(src_hbm, 0, sidx0, ssi0)
    pltpu.async_copy(g_hbm.at[sidx0], rows0, sg0)

    def body(j, carry):
        a = 2 * j
        wait(src_hbm, a + 1, sidx1, ssi1)
        pltpu.async_copy(g_hbm.at[sidx1], rows1, sg1)
        pltpu.make_async_copy(g_hbm.at[sidx0], rows0, sg0).wait()
        load(src_hbm, a + 2, sidx0, ssi0)
        wait(dst_hbm, a, didx0, sdi0)
        pltpu.sync_copy(rows0, acc.at[didx0], add=True)
        load(dst_hbm, a + 2, didx0, sdi0)
        wait(src_hbm, a + 2, sidx0, ssi0)
        pltpu.async_copy(g_hbm.at[sidx0], rows0, sg0)
        pltpu.make_async_copy(g_hbm.at[sidx1], rows1, sg1).wait()
        load(src_hbm, a + 3, sidx1, ssi1)
        wait(dst_hbm, a + 1, didx1, sdi1)
        pltpu.sync_copy(rows1, acc.at[didx1], add=True)
        load(dst_hbm, a + 3, didx1, sdi1)
        return carry

    lax.fori_loop(0, NCHUNK // 2 - 1, body, 0)
    a = NCHUNK - 2
    wait(src_hbm, a + 1, sidx1, ssi1)
    pltpu.async_copy(g_hbm.at[sidx1], rows1, sg1)
    pltpu.make_async_copy(g_hbm.at[sidx0], rows0, sg0).wait()
    wait(dst_hbm, a, didx0, sdi0)
    pltpu.sync_copy(rows0, acc.at[didx0], add=True)
    pltpu.make_async_copy(g_hbm.at[sidx1], rows1, sg1).wait()
    wait(dst_hbm, a + 1, didx1, sdi1)
    pltpu.sync_copy(rows1, acc.at[didx1], add=True)

    plsc.subcore_barrier()
    pltpu.sync_copy(acc.at[pl.ds(s * RPT, RPT)],
                    out_hbm.at[c].at[pl.ds(s * RPT, RPT)])


@functools.partial(
    pl.kernel,
    out_type=jax.ShapeDtypeStruct((NC, NP, DC), jnp.float32),
    mesh=_mesh,
    scratch_types=[
        pltpu.VMEM_SHARED((NP, DC), jnp.float32),
        pltpu.VMEM((NCHUNK, CH), jnp.int32),
        pltpu.VMEM((CH, DC), jnp.float32),
        pltpu.SemaphoreType.DMA,
        pltpu.SemaphoreType.DMA,
    ],
)
def _sc_deg(dst_hbm, zeros_hbm, ones_hbm, out_hbm, acc, didx, ones, s0, s1):
    c = lax.axis_index("c")
    s = lax.axis_index("s")
    t = c * NS + s
    pltpu.sync_copy(zeros_hbm.at[pl.ds(s * RPT, RPT)],
                    acc.at[pl.ds(s * RPT, RPT)])
    pltpu.sync_copy(dst_hbm.at[t], didx)
    pltpu.sync_copy(ones_hbm, ones)
    plsc.subcore_barrier()

    def body(j, carry):
        a = 2 * j
        pltpu.async_copy(ones, acc.at[didx.at[a]], s0, add=True)
        pltpu.async_copy(ones, acc.at[didx.at[a + 1]], s1, add=True)
        pltpu.make_async_copy(ones, acc.at[didx.at[a]], s0).wait()
        pltpu.make_async_copy(ones, acc.at[didx.at[a + 1]], s1).wait()
        return carry

    lax.fori_loop(0, NCHUNK // 2, body, 0)
    plsc.subcore_barrier()
    pltpu.sync_copy(acc.at[pl.ds(s * RPT, RPT)],
                    out_hbm.at[c].at[pl.ds(s * RPT, RPT)])


_RB = 1000


def _tc0_body(degp_ref, x_ref, w_ref, dinv_ref, g_ref):
    deg = degp_ref[0, :, 0:1] + degp_ref[1, :, 0:1] + 1.0
    dinv = lax.rsqrt(deg)
    dinv_ref[...] = dinv
    g_ref[...] = dinv * jnp.dot(x_ref[...], w_ref[...],
                                preferred_element_type=jnp.float32)


def _tc_mid_body(aggp_ref, g_ref, dinv_ref, b_ref, w_ref, gout_ref):
    dinv = dinv_ref[...]
    h = dinv * (aggp_ref[0] + aggp_ref[1] + g_ref[...]) + b_ref[...]
    h = jnp.maximum(h, 0.0)
    gout_ref[...] = dinv * jnp.dot(h, w_ref[...],
                                   preferred_element_type=jnp.float32)


def _tc_fin_body(aggp_ref, g_ref, dinv_ref, b_ref, out_ref):
    out_ref[...] = (dinv_ref[...] * (aggp_ref[0] + aggp_ref[1] + g_ref[...])
                    + b_ref[...])


def _tc0(degp, x, w):
    return pl.pallas_call(
        _tc0_body,
        grid=(N // _RB,),
        in_specs=[
            pl.BlockSpec((NC, _RB, DC), lambda i: (0, i, 0)),
            pl.BlockSpec((_RB, D), lambda i: (i, 0)),
            pl.BlockSpec((D, D), lambda i: (0, 0)),
        ],
        out_specs=[
            pl.BlockSpec((_RB, 1), lambda i: (i, 0)),
            pl.BlockSpec((_RB, D), lambda i: (i, 0)),
        ],
        out_shape=[
            jax.ShapeDtypeStruct((N, 1), jnp.float32),
            jax.ShapeDtypeStruct((N, D), jnp.float32),
        ],
    )(degp, x, w)


def _tc_mid(aggp, g, dinv, b, w):
    return pl.pallas_call(
        _tc_mid_body,
        grid=(N // _RB,),
        in_specs=[
            pl.BlockSpec((NC, _RB, D), lambda i: (0, i, 0)),
            pl.BlockSpec((_RB, D), lambda i: (i, 0)),
            pl.BlockSpec((_RB, 1), lambda i: (i, 0)),
            pl.BlockSpec((1, D), lambda i: (0, 0)),
            pl.BlockSpec((D, D), lambda i: (0, 0)),
        ],
        out_specs=pl.BlockSpec((_RB, D), lambda i: (i, 0)),
        out_shape=jax.ShapeDtypeStruct((N, D), jnp.float32),
    )(aggp, g, dinv, b, w)


def _tc_fin(aggp, g, dinv, b):
    return pl.pallas_call(
        _tc_fin_body,
        grid=(N // _RB,),
        in_specs=[
            pl.BlockSpec((NC, _RB, D), lambda i: (0, i, 0)),
            pl.BlockSpec((_RB, D), lambda i: (i, 0)),
            pl.BlockSpec((_RB, 1), lambda i: (i, 0)),
            pl.BlockSpec((1, D), lambda i: (0, 0)),
        ],
        out_specs=pl.BlockSpec((_RB, D), lambda i: (i, 0)),
        out_shape=jax.ShapeDtypeStruct((N, D), jnp.float32),
    )(aggp, g, dinv, b)


def kernel(x, edge_index, edge_features, W1, b1, W2, b2, W3, b3):
    del edge_features
    src = jnp.pad(edge_index[0].reshape(NW, EPT), ((0, 0), (0, EPTP - EPT)),
                  constant_values=0).reshape(NW, NCHUNK, CH)
    dst = jnp.pad(edge_index[1].reshape(NW, EPT), ((0, 0), (0, EPTP - EPT)),
                  constant_values=PAD_DST).reshape(NW, NCHUNK, CH)
    zeros_d = jnp.zeros((NP, D), jnp.float32)
    zeros_dc = jnp.zeros((NP, DC), jnp.float32)
    ones_dc = jnp.ones((CH, DC), jnp.float32)

    degp = _sc_deg(dst, zeros_dc, ones_dc)
    dinv, g1 = _tc0(degp, x, W1)
    agg1 = _sc_agg(g1, src, dst, zeros_d)
    g2 = _tc_mid(agg1, g1, dinv, b1.reshape(1, D), W2)
    agg2 = _sc_agg(g2, src, dst, zeros_d)
    g3 = _tc_mid(agg2, g2, dinv, b2.reshape(1, D), W3)
    agg3 = _sc_agg(g3, src, dst, zeros_d)
    return _tc_fin(agg3, g3, dinv, b3.reshape(1, D))


# speedup vs baseline: 2.1249x; 1.1116x over previous
import functools

import jax
import jax.numpy as jnp
from jax import lax
from jax.experimental import pallas as pl
from jax.experimental.pallas import tpu as pltpu
from jax.experimental.pallas import tpu_sc as plsc

N = 10000
E = 320000
D = 128
DC = 128
NC = 2
NS = 16
NW = NC * NS
EPT = E // NW
CH = 40
EPTP = 10000
NCHUNK = EPTP // CH
NP = 10240
PAD_DST = 10100
RPT = NP // NS

_mesh = plsc.VectorSubcoreMesh(core_axis_name="c", subcore_axis_name="s")


@functools.partial(
    pl.kernel,
    out_type=jax.ShapeDtypeStruct((NC, NP, D), jnp.float32),
    mesh=_mesh,
    scratch_types=[
        pltpu.VMEM_SHARED((NP, D), jnp.float32),
        pltpu.VMEM((CH,), jnp.int32),
        pltpu.VMEM((CH,), jnp.int32),
        pltpu.VMEM((CH,), jnp.int32),
        pltpu.VMEM((CH,), jnp.int32),
        pltpu.VMEM((CH, D), jnp.float32),
        pltpu.VMEM((CH, D), jnp.float32),
        pltpu.SemaphoreType.DMA,
        pltpu.SemaphoreType.DMA,
        pltpu.SemaphoreType.DMA,
        pltpu.SemaphoreType.DMA,
        pltpu.SemaphoreType.DMA,
        pltpu.SemaphoreType.DMA,
    ],
)
def _sc_agg(g_hbm, src_hbm, dst_hbm, zeros_hbm, out_hbm,
            acc, sidx0, sidx1, didx0, didx1, rows0, rows1,
            ssi0, ssi1, sdi0, sdi1, sg0, sg1):
    c = lax.axis_index("c")
    s = lax.axis_index("s")
    t = c * NS + s

    def load(hbm, k, ref, sem):
        pltpu.async_copy(hbm.at[t, k], ref, sem)

    def wait(hbm, k, ref, sem):
        pltpu.make_async_copy(hbm.at[t, k], ref, sem).wait()

    load(src_hbm, 0, sidx0, ssi0)
    load(dst_hbm, 0, didx0, sdi0)
    load(src_hbm, 1, sidx1, ssi1)
    load(dst_hbm, 1, didx1, sdi1)
    pltpu.sync_copy(zeros_hbm.at[pl.ds(s * RPT, RPT)],
                    acc.at[pl.ds(s * RPT, RPT)])
    plsc.subcore_barrier()
    wait(src_hbm, 0, sidx0, ssi0)
    pltpu.async_copy(g_hbm.at[sidx0], rows0, sg0)

    def body(j, carry):
        a = 2 * j
        wait(src_hbm, a + 1, sidx1, ssi1)
        pltpu.async_copy(g_hbm.at[sidx1], rows1, sg1)
        pltpu.make_async_copy(g_hbm.at[sidx0], rows0, sg0).wait()
        load(src_hbm, a + 2, sidx0, ssi0)
        wait(dst_hbm, a, didx0, sdi0)
        pltpu.sync_copy(rows0, acc.at[didx0], add=True)
        load(dst_hbm, a + 2, didx0, sdi0)
        wait(src_hbm, a + 2, sidx0, ssi0)
        pltpu.async_copy(g_hbm.at[sidx0], rows0, sg0)
        pltpu.make_async_copy(g_hbm.at[sidx1], rows1, sg1).wait()
        load(src_hbm, a + 3, sidx1, ssi1)
        wait(dst_hbm, a + 1, didx1, sdi1)
        pltpu.sync_copy(rows1, acc.at[didx1], add=True)
        load(dst_hbm, a + 3, didx1, sdi1)
        return carry

    lax.fori_loop(0, NCHUNK // 2 - 1, body, 0)
    a = NCHUNK - 2
    wait(src_hbm, a + 1, sidx1, ssi1)
    pltpu.async_copy(g_hbm.at[sidx1], rows1, sg1)
    pltpu.make_async_copy(g_hbm.at[sidx0], rows0, sg0).wait()
    wait(dst_hbm, a, didx0, sdi0)
    pltpu.sync_copy(rows0, acc.at[didx0], add=True)
    pltpu.make_async_copy(g_hbm.at[sidx1], rows1, sg1).wait()
    wait(dst_hbm, a + 1, didx1, sdi1)
    pltpu.sync_copy(rows1, acc.at[didx1], add=True)

    plsc.subcore_barrier()
    pltpu.sync_copy(acc.at[pl.ds(s * RPT, RPT)],
                    out_hbm.at[c].at[pl.ds(s * RPT, RPT)])


@functools.partial(
    pl.kernel,
    out_type=jax.ShapeDtypeStruct((NC, NP, DC), jnp.float32),
    mesh=_mesh,
    scratch_types=[
        pltpu.VMEM_SHARED((NP, DC), jnp.float32),
        pltpu.VMEM((NCHUNK, CH), jnp.int32),
        pltpu.VMEM((CH, DC), jnp.float32),
        pltpu.SemaphoreType.DMA,
        pltpu.SemaphoreType.DMA,
    ],
)
def _sc_deg(dst_hbm, zeros_hbm, ones_hbm, out_hbm, acc, didx, ones, s0, s1):
    c = lax.axis_index("c")
    s = lax.axis_index("s")
    t = c * NS + s
    pltpu.sync_copy(zeros_hbm.at[pl.ds(s * RPT, RPT)],
                    acc.at[pl.ds(s * RPT, RPT)])
    pltpu.sync_copy(dst_hbm.at[t], didx)
    pltpu.sync_copy(ones_hbm, ones)
    plsc.subcore_barrier()

    def body(j, carry):
        a = 2 * j
        pltpu.async_copy(ones, acc.at[didx.at[a]], s0, add=True)
        pltpu.async_copy(ones, acc.at[didx.at[a + 1]], s1, add=True)
        pltpu.make_async_copy(ones, acc.at[didx.at[a]], s0).wait()
        pltpu.make_async_copy(ones, acc.at[didx.at[a + 1]], s1).wait()
        return carry

    lax.fori_loop(0, NCHUNK // 2, body, 0)
    plsc.subcore_barrier()
    pltpu.sync_copy(acc.at[pl.ds(s * RPT, RPT)],
                    out_hbm.at[c].at[pl.ds(s * RPT, RPT)])


_RB = 1000


def _tc0_body(degp_ref, x_ref, w_ref, dinv_ref, g_ref):
    deg = degp_ref[0, :, 0:1] + degp_ref[1, :, 0:1] + 1.0
    dinv = lax.rsqrt(deg)
    dinv_ref[...] = dinv
    g_ref[...] = dinv * jnp.dot(x_ref[...], w_ref[...],
                                preferred_element_type=jnp.float32)


def _tc_mid_body(aggp_ref, g_ref, dinv_ref, b_ref, w_ref, gout_ref):
    dinv = dinv_ref[...]
    h = dinv * (aggp_ref[0] + aggp_ref[1] + g_ref[...]) + b_ref[...]
    h = jnp.maximum(h, 0.0)
    gout_ref[...] = dinv * jnp.dot(h, w_ref[...],
                                   preferred_element_type=jnp.float32)


def _tc_fin_body(aggp_ref, g_ref, dinv_ref, b_ref, out_ref):
    out_ref[...] = (dinv_ref[...] * (aggp_ref[0] + aggp_ref[1] + g_ref[...])
                    + b_ref[...])


def _tc0(degp, x, w):
    return pl.pallas_call(
        _tc0_body,
        grid=(N // _RB,),
        in_specs=[
            pl.BlockSpec((NC, _RB, DC), lambda i: (0, i, 0)),
            pl.BlockSpec((_RB, D), lambda i: (i, 0)),
            pl.BlockSpec((D, D), lambda i: (0, 0)),
        ],
        out_specs=[
            pl.BlockSpec((_RB, 1), lambda i: (i, 0)),
            pl.BlockSpec((_RB, D), lambda i: (i, 0)),
        ],
        out_shape=[
            jax.ShapeDtypeStruct((N, 1), jnp.float32),
            jax.ShapeDtypeStruct((N, D), jnp.float32),
        ],
    )(degp, x, w)


def _tc_mid(aggp, g, dinv, b, w):
    return pl.pallas_call(
        _tc_mid_body,
        grid=(N // _RB,),
        in_specs=[
            pl.BlockSpec((NC, _RB, D), lambda i: (0, i, 0)),
            pl.BlockSpec((_RB, D), lambda i: (i, 0)),
            pl.BlockSpec((_RB, 1), lambda i: (i, 0)),
            pl.BlockSpec((1, D), lambda i: (0, 0)),
            pl.BlockSpec((D, D), lambda i: (0, 0)),
        ],
        out_specs=pl.BlockSpec((_RB, D), lambda i: (i, 0)),
        out_shape=jax.ShapeDtypeStruct((N, D), jnp.float32),
    )(aggp, g, dinv, b, w)


def _tc_fin(aggp, g, dinv, b):
    return pl.pallas_call(
        _tc_fin_body,
        grid=(N // _RB,),
        in_specs=[
            pl.BlockSpec((NC, _RB, D), lambda i: (0, i, 0)),
            pl.BlockSpec((_RB, D), lambda i: (i, 0)),
            pl.BlockSpec((_RB, 1), lambda i: (i, 0)),
            pl.BlockSpec((1, D), lambda i: (0, 0)),
        ],
        out_specs=pl.BlockSpec((_RB, D), lambda i: (i, 0)),
        out_shape=jax.ShapeDtypeStruct((N, D), jnp.float32),
    )(aggp, g, dinv, b)


def kernel(x, edge_index, edge_features, W1, b1, W2, b2, W3, b3):
    del edge_features
    src = jnp.pad(edge_index[0].reshape(NW, EPT), ((0, 0), (0, EPTP - EPT)),
                  constant_values=0).reshape(NW, NCHUNK, CH)
    dst = jnp.pad(edge_index[1].reshape(NW, EPT), ((0, 0), (0, EPTP - EPT)),
                  constant_values=PAD_DST).reshape(NW, NCHUNK, CH)
    zeros_d = jnp.zeros((NP, D), jnp.float32)
    zeros_dc = jnp.zeros((NP, DC), jnp.float32)
    ones_dc = jnp.ones((CH, DC), jnp.float32)

    degp = _sc_deg(dst, zeros_dc, ones_dc)
    dinv, g1 = _tc0(degp, x, W1)
    agg1 = _sc_agg(g1, src, dst, zeros_d)
    g2 = _tc_mid(agg1, g1, dinv, b1.reshape(1, D), W2)
    agg2 = _sc_agg(g2, src, dst, zeros_d)
    g3 = _tc_mid(agg2, g2, dinv, b2.reshape(1, D), W3)
    agg3 = _sc_agg(g3, src, dst, zeros_d)
    return _tc_fin(agg3, g3, dinv, b3.reshape(1, D))
